# baseline (speedup 1.0000x reference)
import functools

import jax
import jax.numpy as jnp
from jax import lax
from jax.experimental import pallas as pl
from jax.experimental.pallas import tpu as pltpu
from jax.experimental.pallas import tpu_sc as plsc

VOCAB = 1000000
DIM = 64
B = 4096
L = 200
EPS = 1e-05

NC = 2
NS = 16
NW = NC * NS
HALF = L // 2
B_PER_W = B // NW


def _rsqrt(x):
    i = jax.lax.bitcast_convert_type(x, jnp.int32)
    i = jnp.int32(0x5F3759DF) - jax.lax.shift_right_arithmetic(i, 1)
    y = jax.lax.bitcast_convert_type(i, jnp.float32)
    for _ in range(2):
        y = y * (1.5 - 0.5 * x * y * y)
    return y


_GATHER_DNUMS = lax.GatherDimensionNumbers(
    offset_dims=(), collapsed_slice_dims=(0,), start_index_map=(0,))


def _lane_shuffle(v, idx):
    return lax.gather(
        v, idx[:, None], dimension_numbers=_GATHER_DNUMS, slice_sizes=(1,),
        mode=lax.GatherScatterMode.PROMISE_IN_BOUNDS)


def _lane_sum(v):
    for sh in (8, 4, 2, 1):
        idx = jax.lax.iota(jnp.int32, 16) ^ sh
        v = v + _lane_shuffle(v, idx)
    return v


def _layernorm_chunk(rows_v, out_v, g_vecs, b_vecs):
    inv_d = 1.0 / DIM

    @plsc.parallel_loop(0, L, step=1, unroll=8)
    def row_body(r):
        v = [rows_v[r, pl.ds(16 * k, 16)] for k in range(4)]
        s = (v[0] + v[1]) + (v[2] + v[3])
        sq = [vk * vk for vk in v]
        t = (sq[0] + sq[1]) + (sq[2] + sq[3])
        mean = _lane_sum(s) * inv_d
        var = _lane_sum(t) * inv_d - mean * mean
        rstd = _rsqrt(var + EPS)
        u = mean * rstd
        for k in range(4):
            out_v[r, pl.ds(16 * k, 16)] = (v[k] * rstd - u) * g_vecs[k] + b_vecs[k]


def _make_sc_kernel():
    mesh = plsc.VectorSubcoreMesh(core_axis_name="c", subcore_axis_name="s")

    @functools.partial(
        pl.kernel,
        out_type=jax.ShapeDtypeStruct((B, L, DIM), jnp.float32),
        mesh=mesh,
        compiler_params=pltpu.CompilerParams(use_tc_tiling_on_sc=False),
        scratch_types=[
            pltpu.VMEM((B_PER_W, 2, HALF), jnp.int32),
            pltpu.VMEM((L, DIM), jnp.float32),
            pltpu.VMEM((L, DIM), jnp.float32),
            pltpu.VMEM((L, DIM), jnp.float32),
            pltpu.VMEM((L, DIM), jnp.float32),
            pltpu.VMEM((DIM,), jnp.float32),
            pltpu.VMEM((DIM,), jnp.float32),
            pltpu.SemaphoreType.DMA,
            pltpu.SemaphoreType.DMA,
            pltpu.SemaphoreType.DMA,
            pltpu.SemaphoreType.DMA,
        ],
    )
    def sc_kernel(x_hbm, table_hbm, gamma_hbm, beta_hbm, out_hbm,
                  idx_v, rows0, rows1, outv0, outv1, gam_v, bet_v,
                  gsem0, gsem1, ssem0, ssem1):
        rows = (rows0, rows1)
        outv = (outv0, outv1)
        gsem = (gsem0, gsem1)
        ssem = (ssem0, ssem1)

        wid = lax.axis_index("s") * NC + lax.axis_index("c")
        b0 = wid * B_PER_W

        pltpu.sync_copy(gamma_hbm, gam_v)
        pltpu.sync_copy(beta_hbm, bet_v)
        g_vecs = [gam_v[pl.ds(16 * k, 16)] for k in range(4)]
        b_vecs = [bet_v[pl.ds(16 * k, 16)] for k in range(4)]

        pltpu.sync_copy(x_hbm.at[pl.ds(b0, B_PER_W)], idx_v)

        def fire_gathers(g, b):
            for h in range(2):
                pltpu.async_copy(
                    table_hbm.at[idx_v.at[g, h]],
                    rows[b].at[pl.ds(h * HALF, HALF)], gsem[b])

        def wait_gathers(b):
            pltpu.make_async_copy(
                table_hbm.at[pl.ds(0, L)], rows[b], gsem[b]).wait()

        def fire_scatter(g, b):
            pltpu.async_copy(outv[b], out_hbm.at[b0 + g], ssem[b])

        def wait_scatter(b):
            pltpu.make_async_copy(
                outv[b], out_hbm.at[0], ssem[b]).wait()

        fire_gathers(0, 0)
        fire_gathers(1, 1)
        for b in (0, 1):
            wait_gathers(b)
            _layernorm_chunk(rows[b], outv[b], g_vecs, b_vecs)
            fire_scatter(b, b)
            fire_gathers(b + 2, b)

        def steady(k, _):
            for b in (0, 1):
                g = 2 * k + b
                wait_gathers(b)
                wait_scatter(b)
                _layernorm_chunk(rows[b], outv[b], g_vecs, b_vecs)
                fire_scatter(g, b)
                fire_gathers(g + 2, b)
            return 0

        lax.fori_loop(1, B_PER_W // 2 - 1, steady, 0)

        for b in (0, 1):
            g = B_PER_W - 2 + b
            wait_gathers(b)
            wait_scatter(b)
            _layernorm_chunk(rows[b], outv[b], g_vecs, b_vecs)
            fire_scatter(g, b)
        for b in (0, 1):
            wait_scatter(b)

    return sc_kernel


_SC_KERNEL = _make_sc_kernel()


def kernel(x, table, gamma, beta):
    x3 = x.reshape(B, 2, HALF)
    return _SC_KERNEL(x3, table, gamma, beta)


# speedup vs baseline: 1.0424x; 1.0424x over previous
import functools

import jax
import jax.numpy as jnp
from jax import lax
from jax.experimental import pallas as pl
from jax.experimental.pallas import tpu as pltpu
from jax.experimental.pallas import tpu_sc as plsc

VOCAB = 1000000
DIM = 64
B = 4096
L = 200
EPS = 1e-05

NC = 2
NS = 16
NW = NC * NS
HALF = L // 2
B_PER_W = B // NW


def _rsqrt(x):
    i = jax.lax.bitcast_convert_type(x, jnp.int32)
    i = jnp.int32(0x5F3759DF) - jax.lax.shift_right_arithmetic(i, 1)
    y = jax.lax.bitcast_convert_type(i, jnp.float32)
    for _ in range(2):
        y = y * (1.5 - 0.5 * x * y * y)
    return y


_GATHER_DNUMS = lax.GatherDimensionNumbers(
    offset_dims=(), collapsed_slice_dims=(0,), start_index_map=(0,))


def _lane_shuffle(v, idx):
    return lax.gather(
        v, idx[:, None], dimension_numbers=_GATHER_DNUMS, slice_sizes=(1,),
        mode=lax.GatherScatterMode.PROMISE_IN_BOUNDS)


def _lane_sum(v):
    for sh in (8, 4, 2, 1):
        idx = jax.lax.iota(jnp.int32, 16) ^ sh
        v = v + _lane_shuffle(v, idx)
    return v


def _layernorm_chunk(rows_v, out_v, g_vecs, b_vecs):
    inv_d = 1.0 / DIM

    @plsc.parallel_loop(0, L, step=1, unroll=8)
    def row_body(r):
        v = [rows_v[r, pl.ds(16 * k, 16)] for k in range(4)]
        s = (v[0] + v[1]) + (v[2] + v[3])
        sq = [vk * vk for vk in v]
        t = (sq[0] + sq[1]) + (sq[2] + sq[3])
        mean = _lane_sum(s) * inv_d
        var = _lane_sum(t) * inv_d - mean * mean
        rstd = _rsqrt(var + EPS)
        u = mean * rstd
        for k in range(4):
            out_v[r, pl.ds(16 * k, 16)] = (v[k] * rstd - u) * g_vecs[k] + b_vecs[k]


def _make_sc_kernel():
    mesh = plsc.VectorSubcoreMesh(core_axis_name="c", subcore_axis_name="s")

    @functools.partial(
        pl.kernel,
        out_type=jax.ShapeDtypeStruct((B, L, DIM), jnp.float32),
        mesh=mesh,
        compiler_params=pltpu.CompilerParams(use_tc_tiling_on_sc=False),
        scratch_types=[
            pltpu.VMEM((B_PER_W, 2, HALF), jnp.int32),
            pltpu.VMEM((L, 128), jnp.float32),
            pltpu.VMEM((L, 128), jnp.float32),
            pltpu.VMEM((L, DIM), jnp.float32),
            pltpu.VMEM((L, DIM), jnp.float32),
            pltpu.VMEM((DIM,), jnp.float32),
            pltpu.VMEM((DIM,), jnp.float32),
            pltpu.SemaphoreType.DMA,
            pltpu.SemaphoreType.DMA,
            pltpu.SemaphoreType.DMA,
            pltpu.SemaphoreType.DMA,
        ],
    )
    def sc_kernel(x_hbm, table_hbm, gamma_hbm, beta_hbm, out_hbm,
                  idx_v, rows0, rows1, outv0, outv1, gam_v, bet_v,
                  gsem0, gsem1, ssem0, ssem1):
        rows = (rows0, rows1)
        outv = (outv0, outv1)
        gsem = (gsem0, gsem1)
        ssem = (ssem0, ssem1)

        wid = lax.axis_index("s") * NC + lax.axis_index("c")
        b0 = wid * B_PER_W

        pltpu.sync_copy(gamma_hbm, gam_v)
        pltpu.sync_copy(beta_hbm, bet_v)
        g_vecs = [gam_v[pl.ds(16 * k, 16)] for k in range(4)]
        b_vecs = [bet_v[pl.ds(16 * k, 16)] for k in range(4)]

        pltpu.sync_copy(x_hbm.at[pl.ds(b0, B_PER_W)], idx_v)

        def fire_gathers(g, b):
            for h in range(2):
                pltpu.async_copy(
                    table_hbm.at[idx_v.at[g, h]],
                    rows[b].at[pl.ds(h * HALF, HALF)], gsem[b])

        def wait_gathers(b):
            pltpu.make_async_copy(
                table_hbm.at[pl.ds(0, L)], rows[b], gsem[b]).wait()

        def fire_scatter(g, b):
            pltpu.async_copy(outv[b], out_hbm.at[b0 + g], ssem[b])

        def wait_scatter(b):
            pltpu.make_async_copy(
                outv[b], out_hbm.at[0], ssem[b]).wait()

        fire_gathers(0, 0)
        fire_gathers(1, 1)
        for b in (0, 1):
            wait_gathers(b)
            _layernorm_chunk(rows[b], outv[b], g_vecs, b_vecs)
            fire_scatter(b, b)
            fire_gathers(b + 2, b)

        def steady(k, _):
            for b in (0, 1):
                g = 2 * k + b
                wait_gathers(b)
                wait_scatter(b)
                _layernorm_chunk(rows[b], outv[b], g_vecs, b_vecs)
                fire_scatter(g, b)
                fire_gathers(g + 2, b)
            return 0

        lax.fori_loop(1, B_PER_W // 2 - 1, steady, 0)

        for b in (0, 1):
            g = B_PER_W - 2 + b
            wait_gathers(b)
            wait_scatter(b)
            _layernorm_chunk(rows[b], outv[b], g_vecs, b_vecs)
            fire_scatter(g, b)
        for b in (0, 1):
            wait_scatter(b)

    return sc_kernel


_SC_KERNEL = _make_sc_kernel()


def kernel(x, table, gamma, beta):
    x3 = x.reshape(B, 2, HALF)
    table128 = jnp.pad(table, ((0, 0), (0, 128 - DIM)))
    return _SC_KERNEL(x3, table128, gamma, beta)


# speedup vs baseline: 1.3270x; 1.2730x over previous
import functools

import jax
import jax.numpy as jnp
from jax import lax
from jax.experimental import pallas as pl
from jax.experimental.pallas import tpu as pltpu
from jax.experimental.pallas import tpu_sc as plsc

VOCAB = 1000000
DIM = 64
B = 4096
L = 200
EPS = 1e-05

NC = 2
NS = 16
NW = NC * NS
BL = B * L
CHUNK = 128
ROWS_PER_W = BL // NW
NCHUNK = ROWS_PER_W // CHUNK


def _rsqrt(x):
    i = jax.lax.bitcast_convert_type(x, jnp.int32)
    i = jnp.int32(0x5F3759DF) - jax.lax.shift_right_arithmetic(i, 1)
    y = jax.lax.bitcast_convert_type(i, jnp.float32)
    for _ in range(2):
        y = y * (1.5 - 0.5 * x * y * y)
    return y


_GATHER_DNUMS = lax.GatherDimensionNumbers(
    offset_dims=(), collapsed_slice_dims=(0,), start_index_map=(0,))


def _lane_shuffle(v, idx):
    return lax.gather(
        v, idx[:, None], dimension_numbers=_GATHER_DNUMS, slice_sizes=(1,),
        mode=lax.GatherScatterMode.PROMISE_IN_BOUNDS)


def _lane_sum(v):
    for sh in (8, 4, 2, 1):
        idx = jax.lax.iota(jnp.int32, 16) ^ sh
        v = v + _lane_shuffle(v, idx)
    return v


def _layernorm_chunk(rows_v, out_v, g_vecs, b_vecs):
    inv_d = 1.0 / DIM

    @plsc.parallel_loop(0, CHUNK, step=1, unroll=8)
    def row_body(r):
        v = [rows_v[r, pl.ds(16 * k, 16)] for k in range(4)]
        s = (v[0] + v[1]) + (v[2] + v[3])
        sq = [vk * vk for vk in v]
        t = (sq[0] + sq[1]) + (sq[2] + sq[3])
        mean = _lane_sum(s) * inv_d
        var = _lane_sum(t) * inv_d - mean * mean
        rstd = _rsqrt(var + EPS)
        u = mean * rstd
        for k in range(4):
            out_v[r, pl.ds(16 * k, 16)] = (v[k] * rstd - u) * g_vecs[k] + b_vecs[k]


def _make_sc_kernel():
    mesh = plsc.VectorSubcoreMesh(core_axis_name="c", subcore_axis_name="s")

    @functools.partial(
        pl.kernel,
        out_type=jax.ShapeDtypeStruct((BL, DIM), jnp.float32),
        mesh=mesh,
        compiler_params=pltpu.CompilerParams(use_tc_tiling_on_sc=True),
        scratch_types=[
            pltpu.VMEM((NCHUNK, CHUNK), jnp.int32),
            pltpu.VMEM((CHUNK, 128), jnp.float32),
            pltpu.VMEM((CHUNK, 128), jnp.float32),
            pltpu.VMEM((CHUNK, DIM), jnp.float32),
            pltpu.VMEM((CHUNK, DIM), jnp.float32),
            pltpu.VMEM((DIM,), jnp.float32),
            pltpu.VMEM((DIM,), jnp.float32),
            pltpu.SemaphoreType.DMA,
            pltpu.SemaphoreType.DMA,
            pltpu.SemaphoreType.DMA,
            pltpu.SemaphoreType.DMA,
        ],
    )
    def sc_kernel(x_hbm, table_hbm, gamma_hbm, beta_hbm, out_hbm,
                  idx_v, rows0, rows1, outv0, outv1, gam_v, bet_v,
                  gsem0, gsem1, ssem0, ssem1):
        rows = (rows0, rows1)
        outv = (outv0, outv1)
        gsem = (gsem0, gsem1)
        ssem = (ssem0, ssem1)

        wid = lax.axis_index("s") * NC + lax.axis_index("c")
        chunk0 = wid * NCHUNK

        pltpu.sync_copy(gamma_hbm, gam_v)
        pltpu.sync_copy(beta_hbm, bet_v)
        g_vecs = [gam_v[pl.ds(16 * k, 16)] for k in range(4)]
        b_vecs = [bet_v[pl.ds(16 * k, 16)] for k in range(4)]

        pltpu.sync_copy(x_hbm.at[pl.ds(chunk0, NCHUNK)], idx_v)

        def fire_gather(g, b):
            pltpu.async_copy(table_hbm.at[idx_v.at[g]], rows[b], gsem[b])

        def wait_gather(b):
            pltpu.make_async_copy(
                table_hbm.at[pl.ds(0, CHUNK)], rows[b], gsem[b]).wait()

        def fire_scatter(g, b):
            dst = out_hbm.at[pl.ds((chunk0 + g) * CHUNK, CHUNK)]
            pltpu.async_copy(outv[b], dst, ssem[b])

        def wait_scatter(b):
            pltpu.make_async_copy(
                outv[b], out_hbm.at[pl.ds(0, CHUNK)], ssem[b]).wait()

        fire_gather(0, 0)
        fire_gather(1, 1)
        for b in (0, 1):
            wait_gather(b)
            _layernorm_chunk(rows[b], outv[b], g_vecs, b_vecs)
            fire_scatter(b, b)
            fire_gather(b + 2, b)

        def steady(k, _):
            for b in (0, 1):
                g = 2 * k + b
                wait_gather(b)
                wait_scatter(b)
                _layernorm_chunk(rows[b], outv[b], g_vecs, b_vecs)
                fire_scatter(g, b)
                fire_gather(g + 2, b)
            return 0

        lax.fori_loop(1, NCHUNK // 2 - 1, steady, 0)

        for b in (0, 1):
            g = NCHUNK - 2 + b
            wait_gather(b)
            wait_scatter(b)
            _layernorm_chunk(rows[b], outv[b], g_vecs, b_vecs)
            fire_scatter(g, b)
        for b in (0, 1):
            wait_scatter(b)

    return sc_kernel


_SC_KERNEL = _make_sc_kernel()


def kernel(x, table, gamma, beta):
    x2 = x.reshape(BL // CHUNK, CHUNK)
    table128 = jnp.pad(table, ((0, 0), (0, 128 - DIM)))
    out = _SC_KERNEL(x2, table128, gamma, beta)
    return out.reshape(B, L, DIM)
